# baseline (speedup 1.0000x reference)
import jax
import jax.numpy as jnp
from jax import lax
from jax.experimental import pallas as pl

_DEPTH = 1000
_L = 20
_BS = 256


def _body(x_ref, o_ref):
    xt = x_ref[...]
    d = lax.broadcasted_iota(jnp.int32, (_L, _DEPTH, _BS), 1)
    o_ref[...] = (xt[:, None, :] == d).astype(jnp.float32)


def kernel(X_in, ones):
    del ones
    B, L = X_in.shape
    XT = X_in.T
    out_phys = pl.pallas_call(
        _body,
        grid=(B // _BS,),
        in_specs=[pl.BlockSpec((L, _BS), lambda i: (0, i))],
        out_specs=pl.BlockSpec((_L, _DEPTH, _BS), lambda i: (0, 0, i)),
        out_shape=jax.ShapeDtypeStruct((L, _DEPTH, B), jnp.float32),
    )(XT)
    return jnp.transpose(out_phys, (2, 1, 0))


# speedup vs baseline: 1.0295x; 1.0295x over previous
import jax
import jax.numpy as jnp
from jax import lax
from jax.experimental import pallas as pl

_DEPTH = 1000
_L = 20
_BS = 128


def _body(x_ref, o_ref):
    xt = x_ref[...]
    d = lax.broadcasted_iota(jnp.int32, (_L, _DEPTH, _BS), 1)
    o_ref[...] = (xt[:, None, :] == d).astype(jnp.float32)


def kernel(X_in, ones):
    del ones
    B, L = X_in.shape
    XT = X_in.T
    out_phys = pl.pallas_call(
        _body,
        grid=(B // _BS,),
        in_specs=[pl.BlockSpec((L, _BS), lambda i: (0, i))],
        out_specs=pl.BlockSpec((_L, _DEPTH, _BS), lambda i: (0, 0, i)),
        out_shape=jax.ShapeDtypeStruct((L, _DEPTH, B), jnp.float32),
    )(XT)
    return jnp.transpose(out_phys, (2, 1, 0))
